# Initial kernel scaffold; baseline (speedup 1.0000x reference)
import jax
import jax.numpy as jnp
from jax.experimental import pallas as pl


def kernel(event_type_ids, prop_vectors, event_type_table, W_prop, b_prop, W_proj, b_proj):
    raise NotImplementedError("write your pallas kernel here")


# speedup vs baseline: 1.3297x; 1.3297x over previous
import functools

import jax
import jax.numpy as jnp
from jax import lax
from jax.experimental import pallas as pl
from jax.experimental.pallas import tpu as pltpu
from jax.experimental.pallas import tpu_sc as plsc

B = 16384
D = 128
V = 100
P = 50

NC, NS = 2, 16
NW = NC * NS
BPW = B // NW

BLK = 2048


def _fuse_body(table_ref, wtop_ref, wbot_ref, bprop_ref, bproj_ref,
               wprop_ref, ftab_ref, wf_ref):
    bias = bprop_ref[...] @ wbot_ref[...] + bproj_ref[...]
    ftab_ref[...] = table_ref[...] @ wtop_ref[...] + bias
    wf_ref[...] = wprop_ref[...] @ wbot_ref[...]


def _fuse_weights(table, wtop, wbot, b_prop, b_proj, w_prop):
    return pl.pallas_call(
        _fuse_body,
        out_shape=(
            jax.ShapeDtypeStruct((V, D), jnp.float32),
            jax.ShapeDtypeStruct((P, D), jnp.float32),
        ),
    )(table, wtop, wbot, b_prop.reshape(1, D), b_proj.reshape(1, D), w_prop)


@functools.cache
def _make_sc_gather():
    mesh = plsc.VectorSubcoreMesh(core_axis_name="c", subcore_axis_name="s",
                                  num_cores=NC, num_subcores=NS)

    @functools.partial(
        pl.kernel,
        out_type=jax.ShapeDtypeStruct((B, D), jnp.float32),
        mesh=mesh,
        scratch_types=[
            pltpu.VMEM((BPW,), jnp.int32),
            pltpu.VMEM((BPW, D), jnp.float32),
            pltpu.SemaphoreType.DMA,
        ],
    )
    def _sc_gather(ftab_hbm, idx_hbm, out_hbm, idx_v, rows_v, sem):
        wid = lax.axis_index("s") * NC + lax.axis_index("c")
        base = wid * BPW
        pltpu.sync_copy(idx_hbm.at[pl.ds(base, BPW)], idx_v)
        pltpu.async_copy(ftab_hbm.at[idx_v], rows_v, sem).wait()
        pltpu.sync_copy(rows_v, out_hbm.at[pl.ds(base, BPW)])

    return _sc_gather


def _combine_body(wf_ref, g_ref, prop_ref, out_ref):
    out_ref[...] = g_ref[...] + prop_ref[...] @ wf_ref[...]


def _combine(w_fused, g, prop):
    grid = B // BLK
    return pl.pallas_call(
        _combine_body,
        grid=(grid,),
        in_specs=[
            pl.BlockSpec((P, D), lambda i: (0, 0)),
            pl.BlockSpec((BLK, D), lambda i: (i, 0)),
            pl.BlockSpec((BLK, P), lambda i: (i, 0)),
        ],
        out_specs=pl.BlockSpec((BLK, D), lambda i: (i, 0)),
        out_shape=jax.ShapeDtypeStruct((B, D), jnp.float32),
    )(w_fused, g, prop)


def kernel(event_type_ids, prop_vectors, event_type_table, W_prop, b_prop,
           W_proj, b_proj):
    ids = event_type_ids.astype(jnp.int32)
    wtop = W_proj[:D]
    wbot = W_proj[D:]
    ftab, w_fused = _fuse_weights(event_type_table, wtop, wbot, b_prop,
                                  b_proj, W_prop)
    g = _make_sc_gather()(ftab, ids)
    return _combine(w_fused, g, prop_vectors)
